# baseline (speedup 1.0000x reference)
import functools

import jax
import jax.numpy as jnp
from jax import lax
from jax.experimental import pallas as pl
from jax.experimental.pallas import tpu as pltpu
from jax.experimental.pallas import tpu_sc as plsc

_NC = 2
_NS = 16
_L = 16
_NW = _NC * _NS


def _sc_argmax(weights_flat, B, E):
    b_per_w = B // _NW
    n_grp = b_per_w // _L
    mesh = plsc.VectorSubcoreMesh(core_axis_name="c", subcore_axis_name="s")

    @functools.partial(
        pl.kernel,
        out_type=jax.ShapeDtypeStruct((B,), jnp.int32),
        mesh=mesh,
        scratch_types=[
            pltpu.VMEM((b_per_w * E,), jnp.float32),
            pltpu.VMEM((b_per_w,), jnp.int32),
        ],
        compiler_params=pltpu.CompilerParams(needs_layout_passes=False),
    )
    def k(w_hbm, out_hbm, w_v, idx_v):
        wid = lax.axis_index("s") * _NC + lax.axis_index("c")
        base = wid * b_per_w

        pltpu.sync_copy(w_hbm.at[pl.ds(base * E, b_per_w * E)], w_v)

        iota = lax.iota(jnp.int32, _L)

        def argmax_group(g, _):
            fvec = (g * _L + iota) * E
            best_v = plsc.load_gather(w_v, [fvec])
            best_i = jnp.zeros((_L,), jnp.int32)
            for e in range(1, E):
                v = plsc.load_gather(w_v, [fvec + e])
                p = v > best_v
                best_v = jnp.where(p, v, best_v)
                best_i = jnp.where(p, e, best_i)
            idx_v[pl.ds(g * _L, _L)] = best_i
            return 0

        lax.fori_loop(0, n_grp, argmax_group, 0)
        pltpu.sync_copy(idx_v, out_hbm.at[pl.ds(base, b_per_w)])

    return k(weights_flat)


def _tc_expand(elems, op_cat, B, E, d):
    BLK = 2048
    NB = B // BLK

    def body(e_ref, t_ref, o_ref):
        e = e_ref[0]
        onehot = (e == lax.broadcasted_iota(jnp.int32, (1, E), 1)
                  ).astype(jnp.float32)
        o_ref[...] = jax.lax.dot_general(
            onehot, t_ref[...],
            dimension_numbers=(((1,), (0,)), ((), ())),
            precision=jax.lax.Precision.HIGHEST,
            preferred_element_type=jnp.float32)

    return pl.pallas_call(
        body,
        grid=(NB,),
        in_specs=[
            pl.BlockSpec((1, BLK, 1), lambda i: (i, 0, 0)),
            pl.BlockSpec((E, d), lambda i: (0, 0)),
        ],
        out_specs=pl.BlockSpec((BLK, d), lambda i: (i, 0)),
        out_shape=jax.ShapeDtypeStruct((B, d), jnp.float32),
    )(elems.reshape(NB, BLK, 1), op_cat)


def kernel(opinions, weights):
    E, B, d = opinions.shape
    op_cat = opinions.reshape(E * B, d)
    elems = _sc_argmax(weights.reshape(B * E), B, E)
    return _tc_expand(elems, op_cat, B, E, d)


# speedup vs baseline: 1.1896x; 1.1896x over previous
import functools

import jax
import jax.numpy as jnp
from jax import lax
from jax.experimental import pallas as pl
from jax.experimental.pallas import tpu as pltpu
from jax.experimental.pallas import tpu_sc as plsc

_NC = 2
_NS = 16
_L = 16
_NW = _NC * _NS


def _sc_argmax(weights_flat, B, E):
    b_per_w = B // _NW
    n_grp = b_per_w // _L
    mesh = plsc.VectorSubcoreMesh(core_axis_name="c", subcore_axis_name="s")

    @functools.partial(
        pl.kernel,
        out_type=jax.ShapeDtypeStruct((B,), jnp.int32),
        mesh=mesh,
        scratch_types=[
            pltpu.VMEM((b_per_w * E,), jnp.float32),
            pltpu.VMEM((b_per_w,), jnp.int32),
        ],
        compiler_params=pltpu.CompilerParams(needs_layout_passes=False),
    )
    def k(w_hbm, out_hbm, w_v, idx_v):
        wid = lax.axis_index("s") * _NC + lax.axis_index("c")
        base = wid * b_per_w

        pltpu.sync_copy(w_hbm.at[pl.ds(base * E, b_per_w * E)], w_v)

        iota = lax.iota(jnp.int32, _L)

        def argmax_group(g, _):
            fvec = (g * _L + iota) * E
            best_v = plsc.load_gather(w_v, [fvec])
            best_i = jnp.zeros((_L,), jnp.int32)
            for e in range(1, E):
                v = plsc.load_gather(w_v, [fvec + e])
                p = v > best_v
                best_v = jnp.where(p, v, best_v)
                best_i = jnp.where(p, e, best_i)
            idx_v[pl.ds(g * _L, _L)] = best_i
            return 0

        lax.fori_loop(0, n_grp, argmax_group, 0)
        pltpu.sync_copy(idx_v, out_hbm.at[pl.ds(base, b_per_w)])

    return k(weights_flat)


def _tc_expand(elems, op_cat, B, E, d):
    BLK = 2048
    NB = B // BLK

    def body(e_ref, t_ref, o_ref):
        e = e_ref[0]
        acc = jnp.broadcast_to(t_ref[0:1, :], (BLK, d))
        for k in range(1, E):
            acc = jnp.where(
                e == k, jnp.broadcast_to(t_ref[k:k + 1, :], (BLK, d)), acc)
        o_ref[...] = acc

    return pl.pallas_call(
        body,
        grid=(NB,),
        in_specs=[
            pl.BlockSpec((1, BLK, 1), lambda i: (i, 0, 0)),
            pl.BlockSpec((E, d), lambda i: (0, 0)),
        ],
        out_specs=pl.BlockSpec((BLK, d), lambda i: (i, 0)),
        out_shape=jax.ShapeDtypeStruct((B, d), jnp.float32),
    )(elems.reshape(NB, BLK, 1), op_cat)


def kernel(opinions, weights):
    E, B, d = opinions.shape
    op_cat = opinions.reshape(E * B, d)
    elems = _sc_argmax(weights.reshape(B * E), B, E)
    return _tc_expand(elems, op_cat, B, E, d)


# speedup vs baseline: 1.5851x; 1.3325x over previous
import functools

import jax
import jax.numpy as jnp
from jax import lax
from jax.experimental import pallas as pl
from jax.experimental.pallas import tpu as pltpu
from jax.experimental.pallas import tpu_sc as plsc

_NC = 2
_NS = 16
_L = 16
_NW = _NC * _NS


def _sc_argmax(weights_flat, B, E):
    b_per_w = B // _NW
    n_grp = b_per_w // _L
    mesh = plsc.VectorSubcoreMesh(core_axis_name="c", subcore_axis_name="s")

    @functools.partial(
        pl.kernel,
        out_type=jax.ShapeDtypeStruct((B,), jnp.int32),
        mesh=mesh,
        scratch_types=[
            pltpu.VMEM((b_per_w * E,), jnp.float32),
            pltpu.VMEM((b_per_w,), jnp.int32),
        ],
        compiler_params=pltpu.CompilerParams(needs_layout_passes=False),
    )
    def k(w_hbm, out_hbm, w_v, idx_v):
        wid = lax.axis_index("s") * _NC + lax.axis_index("c")
        base = wid * b_per_w

        pltpu.sync_copy(w_hbm.at[pl.ds(base * E, b_per_w * E)], w_v)

        iota = lax.iota(jnp.int32, _L)

        def argmax_group(g, _):
            fvec = (g * _L + iota) * E
            best_v = plsc.load_gather(w_v, [fvec])
            best_i = jnp.zeros((_L,), jnp.int32)
            for e in range(1, E):
                v = plsc.load_gather(w_v, [fvec + e])
                p = v > best_v
                best_v = jnp.where(p, v, best_v)
                best_i = jnp.where(p, e, best_i)
            idx_v[pl.ds(g * _L, _L)] = best_i
            return 0

        lax.fori_loop(0, n_grp, argmax_group, 0)
        pltpu.sync_copy(idx_v, out_hbm.at[pl.ds(base, b_per_w)])

    return k(weights_flat)


def _tc_expand(elems, op_cat, B, E, d):
    BLK = 2048
    NB = B // BLK

    def body(e_ref, t_ref, o_ref):
        e = e_ref[0, 0, :].reshape(BLK, 1)
        acc = jnp.broadcast_to(t_ref[0:1, :], (BLK, d))
        for k in range(1, E):
            acc = jnp.where(
                e == k, jnp.broadcast_to(t_ref[k:k + 1, :], (BLK, d)), acc)
        o_ref[...] = acc

    return pl.pallas_call(
        body,
        grid=(NB,),
        in_specs=[
            pl.BlockSpec((1, 1, BLK), lambda i: (i, 0, 0)),
            pl.BlockSpec((E, d), lambda i: (0, 0)),
        ],
        out_specs=pl.BlockSpec((BLK, d), lambda i: (i, 0)),
        out_shape=jax.ShapeDtypeStruct((B, d), jnp.float32),
    )(elems.reshape(NB, 1, BLK), op_cat)


def kernel(opinions, weights):
    E, B, d = opinions.shape
    op_cat = opinions.reshape(E * B, d)
    elems = _sc_argmax(weights.reshape(B * E), B, E)
    return _tc_expand(elems, op_cat, B, E, d)
